# baseline (speedup 1.0000x reference)
import functools

import jax
import jax.numpy as jnp
from jax.experimental import pallas as pl

_N_TOKENS = 16384
_HIDDEN = 2048
_N_EXPERTS = 64
_TOP_K = 8
_BLOCK_N = 2048
_HALF_H = _HIDDEN // 2


def _router_block_kernel(x1_ref, x2_ref, w1_ref, w2_ref, b_ref, logits_ref,
                         weights_ref, idx_ref, mask_ref):
    logits = (
        jax.lax.dot_general(
            x1_ref[...], w1_ref[...],
            dimension_numbers=(((1,), (1,)), ((), ())),
            preferred_element_type=jnp.float32,
        )
        + jax.lax.dot_general(
            x2_ref[...], w2_ref[...],
            dimension_numbers=(((1,), (1,)), ((), ())),
            preferred_element_type=jnp.float32,
        )
        + b_ref[...]
    )
    logits_ref[...] = logits

    nb = logits.shape[0]
    lt = jnp.transpose(logits)
    expert_iota = jax.lax.broadcasted_iota(jnp.int32, (_N_EXPERTS, nb), 0)

    top_vals = []
    top_idx = []
    for _ in range(_TOP_K):
        mx = jnp.max(lt, axis=0, keepdims=True)
        idx = jnp.min(jnp.where(lt == mx, expert_iota, _N_EXPERTS),
                      axis=0, keepdims=True)
        top_vals.append(mx)
        top_idx.append(idx)
        lt = jnp.where(expert_iota == idx, -jnp.inf, lt)

    vals = jnp.concatenate(top_vals, axis=0)
    idx_mat = jnp.concatenate(top_idx, axis=0)
    e = jnp.exp(vals - vals[:1, :])
    w_t = e / jnp.sum(e, axis=0, keepdims=True)
    weights_ref[...] = jnp.transpose(w_t)
    idx_ref[...] = jnp.transpose(idx_mat)

    mask_ref[...] = (
        jax.lax.broadcasted_iota(jnp.int32, (_N_EXPERTS, _TOP_K, nb), 0)
        == idx_mat[None, :, :]
    ).astype(jnp.int32)


@functools.partial(jax.jit)
def _router(x, W, b2):
    n_blocks = _N_TOKENS // _BLOCK_N
    return pl.pallas_call(
        _router_block_kernel,
        grid=(n_blocks,),
        in_specs=[
            pl.BlockSpec((_BLOCK_N, _HALF_H), lambda i: (i, 0)),
            pl.BlockSpec((_BLOCK_N, _HALF_H), lambda i: (i, 1)),
            pl.BlockSpec((_N_EXPERTS, _HALF_H), lambda i: (0, 0)),
            pl.BlockSpec((_N_EXPERTS, _HALF_H), lambda i: (0, 1)),
            pl.BlockSpec((1, _N_EXPERTS), lambda i: (0, 0)),
        ],
        out_specs=[
            pl.BlockSpec((_BLOCK_N, _N_EXPERTS), lambda i: (i, 0)),
            pl.BlockSpec((_BLOCK_N, _TOP_K), lambda i: (i, 0)),
            pl.BlockSpec((_BLOCK_N, _TOP_K), lambda i: (i, 0)),
            pl.BlockSpec((_N_EXPERTS, _TOP_K, _BLOCK_N), lambda i: (0, 0, i)),
        ],
        out_shape=[
            jax.ShapeDtypeStruct((_N_TOKENS, _N_EXPERTS), jnp.float32),
            jax.ShapeDtypeStruct((_N_TOKENS, _TOP_K), jnp.float32),
            jax.ShapeDtypeStruct((_N_TOKENS, _TOP_K), jnp.int32),
            jax.ShapeDtypeStruct((_N_EXPERTS, _TOP_K, _N_TOKENS), jnp.int32),
        ],
    )(x, x, W, W, b2)


def kernel(x, W, b):
    logits, weights, idx, mask = _router(x, W, b.reshape(1, _N_EXPERTS))
    return (logits, weights, idx, mask)


# speedup vs baseline: 1.2517x; 1.2517x over previous
import functools

import jax
import jax.numpy as jnp
from jax.experimental import pallas as pl

_N_TOKENS = 16384
_HIDDEN = 2048
_N_EXPERTS = 64
_TOP_K = 8
_BLOCK_N = 2048


def _router_block_kernel(x_ref, w_ref, b_ref, logits_ref, weights_ref,
                         idx_ref, mask_ref):
    x = x_ref[...]
    w = w_ref[...]
    logits = jax.lax.dot_general(
        x, w,
        dimension_numbers=(((1,), (1,)), ((), ())),
        preferred_element_type=jnp.float32,
    ) + b_ref[...]
    logits_ref[...] = logits

    nb = logits.shape[0]
    lt = jnp.transpose(logits)
    expert_iota = jax.lax.broadcasted_iota(jnp.int32, (_N_EXPERTS, nb), 0)

    top_vals = []
    top_idx = []
    for _ in range(_TOP_K):
        mx = jnp.max(lt, axis=0, keepdims=True)
        idx = jnp.min(jnp.where(lt == mx, expert_iota, _N_EXPERTS),
                      axis=0, keepdims=True)
        top_vals.append(mx)
        top_idx.append(idx)
        lt = jnp.where(expert_iota == idx, -jnp.inf, lt)

    vals = jnp.concatenate(top_vals, axis=0)
    idx_mat = jnp.concatenate(top_idx, axis=0)
    e = jnp.exp(vals - vals[:1, :])
    weights_ref[...] = e / jnp.sum(e, axis=0, keepdims=True)
    idx_ref[...] = idx_mat

    mask_ref[...] = (
        jax.lax.broadcasted_iota(jnp.int32, (_N_EXPERTS, _TOP_K, nb), 0)
        == idx_mat[None, :, :]
    ).astype(jnp.int32)


@functools.partial(jax.jit)
def _router(x, W, b2):
    n_blocks = _N_TOKENS // _BLOCK_N
    return pl.pallas_call(
        _router_block_kernel,
        grid=(n_blocks,),
        in_specs=[
            pl.BlockSpec((_BLOCK_N, _HIDDEN), lambda i: (i, 0)),
            pl.BlockSpec((_N_EXPERTS, _HIDDEN), lambda i: (0, 0)),
            pl.BlockSpec((1, _N_EXPERTS), lambda i: (0, 0)),
        ],
        out_specs=[
            pl.BlockSpec((_BLOCK_N, _N_EXPERTS), lambda i: (i, 0)),
            pl.BlockSpec((_TOP_K, _BLOCK_N), lambda i: (0, i)),
            pl.BlockSpec((_TOP_K, _BLOCK_N), lambda i: (0, i)),
            pl.BlockSpec((_N_EXPERTS, _TOP_K, _BLOCK_N), lambda i: (0, 0, i)),
        ],
        out_shape=[
            jax.ShapeDtypeStruct((_N_TOKENS, _N_EXPERTS), jnp.float32),
            jax.ShapeDtypeStruct((_TOP_K, _N_TOKENS), jnp.float32),
            jax.ShapeDtypeStruct((_TOP_K, _N_TOKENS), jnp.int32),
            jax.ShapeDtypeStruct((_N_EXPERTS, _TOP_K, _N_TOKENS), jnp.int32),
        ],
    )(x, W, b2)


def kernel(x, W, b):
    logits, weights_t, idx_t, mask = _router(x, W, b.reshape(1, _N_EXPERTS))
    return (logits, jnp.transpose(weights_t), jnp.transpose(idx_t), mask)


# speedup vs baseline: 1.2529x; 1.0010x over previous
import functools

import jax
import jax.numpy as jnp
from jax.experimental import pallas as pl

_N_TOKENS = 16384
_HIDDEN = 2048
_N_EXPERTS = 64
_TOP_K = 8
_BLOCK_N = 2048


def _router_block_kernel(xa_ref, xb_ref, w_ref, b_ref, logits_ref,
                         weights_ref, idx_ref, mask_ref):
    w = w_ref[...]
    dims = (((1,), (1,)), ((), ()))
    logits = jnp.concatenate(
        [
            jax.lax.dot_general(xa_ref[...], w, dimension_numbers=dims,
                                preferred_element_type=jnp.float32),
            jax.lax.dot_general(xb_ref[...], w, dimension_numbers=dims,
                                preferred_element_type=jnp.float32),
        ],
        axis=0,
    ) + b_ref[...]
    logits_ref[...] = logits

    nb = logits.shape[0]
    lt = jnp.transpose(logits)
    expert_iota = jax.lax.broadcasted_iota(jnp.int32, (_N_EXPERTS, nb), 0)

    top_vals = []
    top_idx = []
    for _ in range(_TOP_K):
        mx = jnp.max(lt, axis=0, keepdims=True)
        idx = jnp.min(jnp.where(lt == mx, expert_iota, _N_EXPERTS),
                      axis=0, keepdims=True)
        top_vals.append(mx)
        top_idx.append(idx)
        lt = jnp.where(expert_iota == idx, -jnp.inf, lt)

    vals = jnp.concatenate(top_vals, axis=0)
    idx_mat = jnp.concatenate(top_idx, axis=0)
    e = jnp.exp(vals - vals[:1, :])
    weights_ref[...] = e / jnp.sum(e, axis=0, keepdims=True)
    idx_ref[...] = idx_mat

    mask_ref[...] = (
        jax.lax.broadcasted_iota(jnp.int32, (_N_EXPERTS, _TOP_K, nb), 0)
        == idx_mat[None, :, :]
    ).astype(jnp.int32)


@functools.partial(jax.jit)
def _router(x, W, b2):
    n_blocks = _N_TOKENS // _BLOCK_N
    return pl.pallas_call(
        _router_block_kernel,
        grid=(n_blocks,),
        in_specs=[
            pl.BlockSpec((_BLOCK_N // 2, _HIDDEN), lambda i: (2 * i, 0)),
            pl.BlockSpec((_BLOCK_N // 2, _HIDDEN), lambda i: (2 * i + 1, 0)),
            pl.BlockSpec((_N_EXPERTS, _HIDDEN), lambda i: (0, 0)),
            pl.BlockSpec((1, _N_EXPERTS), lambda i: (0, 0)),
        ],
        out_specs=[
            pl.BlockSpec((_BLOCK_N, _N_EXPERTS), lambda i: (i, 0)),
            pl.BlockSpec((_TOP_K, _BLOCK_N), lambda i: (0, i)),
            pl.BlockSpec((_TOP_K, _BLOCK_N), lambda i: (0, i)),
            pl.BlockSpec((_N_EXPERTS, _TOP_K, _BLOCK_N), lambda i: (0, 0, i)),
        ],
        out_shape=[
            jax.ShapeDtypeStruct((_N_TOKENS, _N_EXPERTS), jnp.float32),
            jax.ShapeDtypeStruct((_TOP_K, _N_TOKENS), jnp.float32),
            jax.ShapeDtypeStruct((_TOP_K, _N_TOKENS), jnp.int32),
            jax.ShapeDtypeStruct((_N_EXPERTS, _TOP_K, _N_TOKENS), jnp.int32),
        ],
    )(x, x, W, b2)


def kernel(x, W, b):
    logits, weights_t, idx_t, mask = _router(x, W, b.reshape(1, _N_EXPERTS))
    return (logits, jnp.transpose(weights_t), jnp.transpose(idx_t), mask)
